# baseline (speedup 1.0000x reference)
import functools

import jax
import jax.numpy as jnp
from jax import lax
from jax.experimental import pallas as pl
from jax.experimental.pallas import tpu as pltpu
from jax.experimental.pallas import tpu_sc as plsc

_T, _N = 2048, 32768
_BT = 64
_NW = 32
_T_SC = 512
_RPW = _T_SC // _NW
_T_TC = _T - _T_SC
_L = 16


def _tc_body(w_ref, x_ref, o_ref, sw_ref):
    i = pl.program_id(0)

    @pl.when(i == 0)
    def _():
        wv = w_ref[...]
        m = jnp.max(wv)
        e = jnp.exp(wv - m)
        sw_ref[...] = e / jnp.sum(e)

    o_ref[...] = jnp.sum(x_ref[...] * sw_ref[...], axis=1, keepdims=True)


_sc_mesh = plsc.VectorSubcoreMesh(core_axis_name="c", subcore_axis_name="s")


@functools.partial(
    pl.kernel,
    mesh=_sc_mesh,
    out_type=jax.ShapeDtypeStruct((_T_SC,), jnp.float32),
    scratch_types=[
        pltpu.VMEM((_N,), jnp.float32),
        pltpu.VMEM((_N,), jnp.float32),
        pltpu.VMEM((_RPW,), jnp.float32),
    ],
)
def _sc_matvec(x_flat, w_hbm, o_hbm, ebuf, xbuf, ybuf):
    wid = lax.axis_index("s") * 2 + lax.axis_index("c")
    nchunks = _N // _L
    lane = lax.iota(jnp.int32, _L)

    def allreduce(v, op):
        for s in (1, 2, 4, 8):
            v = op(v, v.at[lane ^ s].get(mode="promise_in_bounds"))
        return v

    pltpu.sync_copy(w_hbm, ebuf)

    def mx_body(i, acc):
        return jnp.maximum(acc, ebuf[pl.ds(i * _L, _L)])

    m16 = allreduce(
        lax.fori_loop(0, nchunks, mx_body,
                      jnp.full((_L,), -jnp.inf, jnp.float32)),
        jnp.maximum)

    def ex_body(i, s):
        v = jnp.exp(ebuf[pl.ds(i * _L, _L)] - m16)
        ebuf[pl.ds(i * _L, _L)] = v
        return s + v

    s16 = lax.fori_loop(0, nchunks, ex_body, jnp.zeros((_L,), jnp.float32))
    inv_total = 1.0 / allreduce(s16, jnp.add)

    row0 = _T_TC + wid * _RPW
    yvec = jnp.zeros((_L,), jnp.float32)
    for r in range(_RPW):
        pltpu.sync_copy(x_flat.at[pl.ds((row0 + r) * _N, _N)], xbuf)

        def dot_body(i, acc):
            return acc + xbuf[pl.ds(i * _L, _L)] * ebuf[pl.ds(i * _L, _L)]

        a16 = lax.fori_loop(0, nchunks, dot_body,
                            jnp.zeros((_L,), jnp.float32))
        yvec = jnp.where(lane == r, allreduce(a16, jnp.add) * inv_total, yvec)

    ybuf[...] = yvec
    pltpu.sync_copy(ybuf, o_hbm.at[pl.ds(wid * _RPW, _RPW)])


def kernel(x, w, k):
    del k
    t, n = x.shape

    y_tc = pl.pallas_call(
        _tc_body,
        grid=(_T_TC // _BT,),
        in_specs=[
            pl.BlockSpec((1, n), lambda i: (0, 0)),
            pl.BlockSpec((_BT, n), lambda i: (i, 0)),
        ],
        out_specs=pl.BlockSpec((_BT, 1), lambda i: (i, 0)),
        out_shape=jax.ShapeDtypeStruct((_T_TC, 1), jnp.float32),
        scratch_shapes=[pltpu.VMEM((1, n), jnp.float32)],
    )(w.reshape(1, n), x)

    y_sc = _sc_matvec(x.reshape(-1), w)
    return jnp.concatenate([y_tc.reshape(_T_TC), y_sc])


# speedup vs baseline: 3.0005x; 3.0005x over previous
import functools

import jax
import jax.numpy as jnp
from jax import lax
from jax.experimental import pallas as pl
from jax.experimental.pallas import tpu as pltpu
from jax.experimental.pallas import tpu_sc as plsc

_T, _N = 2048, 32768
_BT = 64
_NW = 32
_T_SC = 512
_RPW = _T_SC // _NW
_T_TC = _T - _T_SC
_L = 16


def _tc_body(w_ref, x_ref, o_ref, sw_ref):
    i = pl.program_id(0)

    @pl.when(i == 0)
    def _():
        wv = w_ref[...]
        m = jnp.max(wv)
        e = jnp.exp(wv - m)
        sw_ref[...] = e / jnp.sum(e)

    o_ref[...] = jnp.sum(x_ref[...] * sw_ref[...], axis=1, keepdims=True)


_sc_mesh = plsc.VectorSubcoreMesh(core_axis_name="c", subcore_axis_name="s")


@functools.partial(
    pl.kernel,
    mesh=_sc_mesh,
    out_type=jax.ShapeDtypeStruct((_T_SC,), jnp.float32),
    scratch_types=[
        pltpu.VMEM((_N,), jnp.float32),
        pltpu.VMEM((_N,), jnp.float32),
        pltpu.VMEM((_RPW,), jnp.float32),
    ],
)
def _sc_matvec(x_hbm, w_hbm, o_hbm, ebuf, xbuf, ybuf):
    wid = lax.axis_index("s") * 2 + lax.axis_index("c")
    nchunks = _N // _L
    lane = lax.iota(jnp.int32, _L)

    def allreduce(v, op):
        for s in (1, 2, 4, 8):
            v = op(v, v.at[lane ^ s].get(mode="promise_in_bounds"))
        return v

    pltpu.sync_copy(w_hbm, ebuf)

    def mx_body(i, acc):
        return jnp.maximum(acc, ebuf[pl.ds(i * _L, _L)])

    m16 = allreduce(
        lax.fori_loop(0, nchunks, mx_body,
                      jnp.full((_L,), -jnp.inf, jnp.float32)),
        jnp.maximum)

    def ex_body(i, s):
        v = jnp.exp(ebuf[pl.ds(i * _L, _L)] - m16)
        ebuf[pl.ds(i * _L, _L)] = v
        return s + v

    s16 = lax.fori_loop(0, nchunks, ex_body, jnp.zeros((_L,), jnp.float32))
    inv_total = 1.0 / allreduce(s16, jnp.add)

    row0 = _T_TC + wid * _RPW
    yvec = jnp.zeros((_L,), jnp.float32)
    _U = 8
    for r in range(_RPW):
        pltpu.sync_copy(x_hbm.at[row0 + r], xbuf)

        def dot_body(i, accs):
            accs = list(accs)
            base = i * (_U * _L)
            for j in range(_U):
                o = base + j * _L
                accs[j % 4] = accs[j % 4] + (
                    xbuf[pl.ds(o, _L)] * ebuf[pl.ds(o, _L)])
            return tuple(accs)

        z = jnp.zeros((_L,), jnp.float32)
        a0, a1, a2, a3 = lax.fori_loop(0, nchunks // _U, dot_body,
                                       (z, z, z, z))
        a16 = (a0 + a1) + (a2 + a3)
        yvec = jnp.where(lane == r, allreduce(a16, jnp.add) * inv_total, yvec)

    ybuf[...] = yvec
    pltpu.sync_copy(ybuf, o_hbm.at[pl.ds(wid * _RPW, _RPW)])


def kernel(x, w, k):
    del k
    t, n = x.shape

    y_tc = pl.pallas_call(
        _tc_body,
        grid=(_T_TC // _BT,),
        in_specs=[
            pl.BlockSpec((1, n), lambda i: (0, 0)),
            pl.BlockSpec((_BT, n), lambda i: (i, 0)),
        ],
        out_specs=pl.BlockSpec((_BT, 1), lambda i: (i, 0)),
        out_shape=jax.ShapeDtypeStruct((_T_TC, 1), jnp.float32),
        scratch_shapes=[pltpu.VMEM((1, n), jnp.float32)],
    )(w.reshape(1, n), x)

    y_sc = _sc_matvec(x, w)
    return jnp.concatenate([y_tc.reshape(_T_TC), y_sc])
